# Initial kernel scaffold; baseline (speedup 1.0000x reference)
import jax
import jax.numpy as jnp
from jax.experimental import pallas as pl


def kernel(attention_mask, target, img_batch_shape):
    raise NotImplementedError("write your pallas kernel here")


# speedup vs baseline: 10.4204x; 10.4204x over previous
import functools

import jax
import jax.numpy as jnp
from jax import lax
from jax.experimental import pallas as pl
from jax.experimental.pallas import tpu as pltpu


def _tc_body(hw_ref, tgt_ref, am_ref, out_ref, *, B, AH, AW, N):
    h = hw_ref[0]
    w = hw_ref[1]
    imgid = tgt_ref[0:1, :]
    x = tgt_ref[2:3, :]
    y = tgt_ref[3:4, :]
    bw = tgt_ref[4:5, :]
    bh = tgt_ref[5:6, :]
    bx1 = w * (x - bw * 0.5)
    by1 = h * (y - bh * 0.5)
    bx2 = w * (x + bw * 0.5)
    by2 = h * (y + bh * 0.5)
    cond = (bx1 <= w) & (by1 <= h) & (bx2 <= w) & (by2 <= h)
    sx = jnp.float32(AW) / w
    sy = jnp.float32(AH) / h
    x1 = jnp.maximum(jnp.trunc(bx1 * sx), 0.0)
    y1 = jnp.maximum(jnp.trunc(by1 * sy), 0.0)
    x2 = jnp.minimum(jnp.ceil(bx2 * sx) + 1.0, jnp.float32(AW))
    y2 = jnp.minimum(jnp.ceil(by2 * sy) + 1.0, jnp.float32(AH))

    rowi = lax.broadcasted_iota(jnp.int32, (AH, N), 0).astype(jnp.float32)
    rowm = ((rowi >= y1) & (rowi < y2)).astype(jnp.float32)
    coli = lax.broadcasted_iota(jnp.int32, (AW, N), 0).astype(jnp.float32)
    colm = ((coli >= x1) & (coli < x2)).astype(jnp.float32)
    jidx = lax.broadcasted_iota(jnp.int32, (B, N), 0).astype(jnp.float32)
    belongs = jidx == imgid
    valid = (belongs & cond).astype(jnp.float32)

    a = (valid[:, None, :] * rowm[None, :, :]).reshape(B * AH, N)
    counts = lax.dot_general(a, colm, (((1,), (1,)), ((), ())),
                             preferred_element_type=jnp.float32)
    gt = (counts > 0).astype(jnp.float32)

    am = am_ref[...]
    sel = (am >= 0).astype(jnp.float32)
    per = jnp.maximum(am, 0.0) - am * gt + jnp.log1p(jnp.exp(-jnp.abs(am)))
    row_l = jnp.sum(per * sel, axis=1, keepdims=True)
    row_s = jnp.sum(sel, axis=1, keepdims=True)
    rows2 = jnp.concatenate([row_l, row_s], axis=1)
    seg = (lax.broadcasted_iota(jnp.int32, (B, B * AH), 1) // AH
           == lax.broadcasted_iota(jnp.int32, (B, B * AH), 0)
           ).astype(jnp.float32)
    nm = lax.dot_general(seg, rows2, (((1,), (0,)), ((), ())),
                         preferred_element_type=jnp.float32)
    num = nm[:, 0:1]
    den = nm[:, 1:2]
    has = jnp.max(belongs.astype(jnp.float32), axis=1, keepdims=True)
    out_ref[0, 0] = jnp.sum(jnp.where(has > 0, num / den, 0.0))


def kernel(attention_mask, target, img_batch_shape):
    B, _, AH, AW = attention_mask.shape
    N = target.shape[0]
    if N == 0:
        return jnp.float32(0.0)
    hw = jnp.asarray(img_batch_shape).astype(jnp.float32)[2:4]
    tgt = jnp.transpose(target.astype(jnp.float32))
    am = attention_mask.reshape(B * AH, AW)
    out = pl.pallas_call(
        functools.partial(_tc_body, B=B, AH=AH, AW=AW, N=N),
        in_specs=[
            pl.BlockSpec(memory_space=pltpu.SMEM),
            pl.BlockSpec(memory_space=pltpu.VMEM),
            pl.BlockSpec(memory_space=pltpu.VMEM),
        ],
        out_specs=pl.BlockSpec(memory_space=pltpu.SMEM),
        out_shape=jax.ShapeDtypeStruct((1, 1), jnp.float32),
    )(hw, tgt, am)
    return out[0, 0]
